# baseline (speedup 1.0000x reference)
import functools

import jax
import jax.numpy as jnp
from jax import lax
from jax.experimental import pallas as pl
from jax.experimental.pallas import tpu as pltpu
from jax.experimental.pallas import tpu_sc as plsc

EMBEDDING_DIM = 128
NUM_CORES = 2
NUM_SUBCORES = 16
NUM_WORKERS = NUM_CORES * NUM_SUBCORES
CHUNK = 128
NBUF = 5

TC_ROWS = 40960
TC_BLOCK = 512


@functools.lru_cache(maxsize=None)
def _make_sc_gather(n_rows: int, out_rows: int, dim: int):
    assert n_rows % (NUM_WORKERS * CHUNK * NBUF) == 0
    rows_per_w = n_rows // NUM_WORKERS
    n_chunks = rows_per_w // CHUNK
    n_groups = n_chunks // NBUF

    mesh = plsc.VectorSubcoreMesh(
        core_axis_name="c", subcore_axis_name="s",
        num_cores=NUM_CORES, num_subcores=NUM_SUBCORES)

    @functools.partial(
        pl.kernel,
        out_type=jax.ShapeDtypeStruct((out_rows, dim), jnp.float32),
        mesh=mesh,
        scratch_types=[
            pltpu.VMEM((n_chunks, CHUNK), jnp.int32),
            pltpu.VMEM((NBUF, CHUNK, dim), jnp.float32),
            pltpu.SemaphoreType.DMA,
            pltpu.SemaphoreType.DMA((NBUF,)),
            pltpu.SemaphoreType.DMA((NBUF,)),
        ],
    )
    def sc_gather(idx_hbm, table_hbm, out_hbm, idx_v, buf, isem, gsem,
                  wsem):
        wid = lax.axis_index("s") * NUM_CORES + lax.axis_index("c")
        base = wid * rows_per_w

        pltpu.async_copy(idx_hbm.at[wid], idx_v, isem).wait()

        def gstart(b, c):
            pltpu.async_copy(table_hbm.at[idx_v.at[c]], buf.at[b],
                             gsem.at[b])

        def gwait(b):
            pltpu.make_async_copy(
                table_hbm.at[idx_v.at[0]], buf.at[b], gsem.at[b]).wait()

        def out_slice(c):
            return out_hbm.at[pl.ds(base + c * CHUNK, CHUNK)]

        def wstart(b, c):
            pltpu.async_copy(buf.at[b], out_slice(c), wsem.at[b])

        def wwait(b):
            pltpu.make_async_copy(buf.at[b], out_slice(0), wsem.at[b]).wait()

        for b in range(NBUF):
            gstart(b, b)

        @pl.loop(0, n_groups)
        def _group(g):
            c0 = g * NBUF
            for b in range(NBUF):
                gwait(b)
                wstart(b, c0 + b)

            @pl.when(g < n_groups - 1)
            def _next():
                for b in range(NBUF):
                    wwait(b)
                    gstart(b, c0 + NBUF + b)

        for b in range(NBUF):
            wwait(b)

    return sc_gather


def _tc_gather_body(idx_sref, table_ref, out_ref):
    def row(i, carry):
        r = idx_sref[i]
        out_ref[pl.ds(i, 1), :] = table_ref[pl.ds(r, 1), :]
        return carry

    lax.fori_loop(0, TC_BLOCK, row, 0, unroll=8)


@functools.lru_cache(maxsize=None)
def _make_tc_gather(n_rows: int, n_table: int, dim: int):
    assert n_rows % TC_BLOCK == 0
    grid = (n_rows // TC_BLOCK,)
    return pl.pallas_call(
        _tc_gather_body,
        grid=grid,
        in_specs=[
            pl.BlockSpec((TC_BLOCK,), lambda i: (i,),
                         memory_space=pltpu.SMEM),
            pl.BlockSpec((n_table, dim), lambda i: (0, 0)),
        ],
        out_specs=pl.BlockSpec((TC_BLOCK, dim), lambda i: (i, 0)),
        out_shape=jax.ShapeDtypeStruct((n_rows, dim), jnp.float32),
    )


def kernel(indices, weight):
    n_rows = indices.size
    dim = weight.shape[-1]
    flat_idx = indices.reshape(-1)
    sc_rows = n_rows - TC_ROWS
    sc_idx = flat_idx[:sc_rows].reshape(
        NUM_WORKERS, sc_rows // (NUM_WORKERS * CHUNK), CHUNK)
    sc_out = _make_sc_gather(sc_rows, n_rows, dim)(sc_idx, weight)
    tc_out = _make_tc_gather(TC_ROWS, weight.shape[0], dim)(
        flat_idx[sc_rows:], weight)
    out = lax.dynamic_update_slice(sc_out, tc_out, (sc_rows, 0))
    return out.reshape(indices.shape + (dim,)).astype(jnp.float32)


# speedup vs baseline: 1.1979x; 1.1979x over previous
import functools

import jax
import jax.numpy as jnp
from jax import lax
from jax.experimental import pallas as pl
from jax.experimental.pallas import tpu as pltpu
from jax.experimental.pallas import tpu_sc as plsc

EMBEDDING_DIM = 128
NUM_CORES = 2
NUM_SUBCORES = 16
NUM_WORKERS = NUM_CORES * NUM_SUBCORES
CHUNK = 128
NBUF = 5


@functools.lru_cache(maxsize=None)
def _make_gather(n_rows: int, dim: int):
    assert n_rows % (NUM_WORKERS * CHUNK) == 0
    rows_per_w = n_rows // NUM_WORKERS
    n_chunks = rows_per_w // CHUNK
    assert n_chunks % NBUF == 0
    n_groups = n_chunks // NBUF

    mesh = plsc.VectorSubcoreMesh(
        core_axis_name="c", subcore_axis_name="s",
        num_cores=NUM_CORES, num_subcores=NUM_SUBCORES)

    @functools.partial(
        pl.kernel,
        out_type=jax.ShapeDtypeStruct((n_rows, dim), jnp.float32),
        mesh=mesh,
        scratch_types=[
            pltpu.VMEM((n_chunks, CHUNK), jnp.int32),
            pltpu.VMEM((NBUF, CHUNK, dim), jnp.float32),
            pltpu.SemaphoreType.DMA,
            pltpu.SemaphoreType.DMA((NBUF,)),
            pltpu.SemaphoreType.DMA((NBUF,)),
        ],
    )
    def gather_kernel(idx_hbm, table_hbm, out_hbm, idx_v, buf, isem,
                      gsem, wsem):
        wid = lax.axis_index("s") * NUM_CORES + lax.axis_index("c")
        base = wid * rows_per_w

        pltpu.async_copy(idx_hbm.at[wid], idx_v, isem).wait()

        def gstart(b, c):
            pltpu.async_copy(table_hbm.at[idx_v.at[c]], buf.at[b],
                             gsem.at[b])

        def gwait(b):
            pltpu.make_async_copy(
                table_hbm.at[idx_v.at[0]], buf.at[b], gsem.at[b]).wait()

        def out_slice(c):
            return out_hbm.at[pl.ds(base + c * CHUNK, CHUNK)]

        def wstart(b, c):
            pltpu.async_copy(buf.at[b], out_slice(c), wsem.at[b])

        def wwait(b):
            pltpu.make_async_copy(buf.at[b], out_slice(0), wsem.at[b]).wait()

        for b in range(NBUF):
            gstart(b, b)

        @pl.loop(0, n_groups)
        def _group(g):
            c0 = g * NBUF
            for b in range(NBUF):
                gwait(b)
                wstart(b, c0 + b)

            @pl.when(g < n_groups - 1)
            def _next():
                for b in range(NBUF):
                    wwait(b)
                    gstart(b, c0 + NBUF + b)

        for b in range(NBUF):
            wwait(b)

    return gather_kernel


def kernel(indices, weight):
    n_rows = indices.size
    dim = weight.shape[-1]
    idx_grouped = indices.reshape(NUM_WORKERS, n_rows // (NUM_WORKERS * CHUNK),
                                  CHUNK)
    out = _make_gather(n_rows, dim)(idx_grouped, weight)
    return out.reshape(indices.shape + (dim,)).astype(jnp.float32)


# speedup vs baseline: 2.1218x; 1.7712x over previous
import functools

import jax
import jax.numpy as jnp
from jax import lax
from jax.experimental import pallas as pl
from jax.experimental.pallas import tpu as pltpu
from jax.experimental.pallas import tpu_sc as plsc

NUM_CORES = 2
NUM_SUBCORES = 16
NUM_WORKERS = NUM_CORES * NUM_SUBCORES
BATCHES_PER_CHUNK = 2
NBUF = 4


@functools.lru_cache(maxsize=None)
def _make_gather(n_batch: int, seq: int, dim: int):
    rows_per_chunk = BATCHES_PER_CHUNK * seq
    assert n_batch % (NUM_WORKERS * BATCHES_PER_CHUNK) == 0
    batches_per_w = n_batch // NUM_WORKERS
    n_chunks = batches_per_w // BATCHES_PER_CHUNK
    assert n_chunks % NBUF == 0
    n_groups = n_chunks // NBUF

    mesh = plsc.VectorSubcoreMesh(
        core_axis_name="c", subcore_axis_name="s",
        num_cores=NUM_CORES, num_subcores=NUM_SUBCORES)

    @functools.partial(
        pl.kernel,
        out_type=jax.ShapeDtypeStruct((n_batch, seq, dim), jnp.float32),
        mesh=mesh,
        scratch_types=[
            pltpu.VMEM((n_chunks, rows_per_chunk), jnp.int32),
            pltpu.VMEM((NBUF, rows_per_chunk, dim), jnp.float32),
            pltpu.SemaphoreType.DMA,
            pltpu.SemaphoreType.DMA((NBUF,)),
            pltpu.SemaphoreType.DMA((NBUF,)),
        ],
    )
    def gather_kernel(idx_hbm, table_hbm, out_hbm, idx_v, buf, isem,
                      gsem, wsem):
        wid = lax.axis_index("s") * NUM_CORES + lax.axis_index("c")
        base = wid * batches_per_w

        pltpu.async_copy(idx_hbm.at[wid], idx_v, isem).wait()

        def gstart(b, c):
            pltpu.async_copy(table_hbm.at[idx_v.at[c]], buf.at[b],
                             gsem.at[b])

        def gwait(b):
            pltpu.make_async_copy(
                table_hbm.at[idx_v.at[0]], buf.at[b], gsem.at[b]).wait()

        def wstart(b, c):
            for r in range(BATCHES_PER_CHUNK):
                pltpu.async_copy(
                    buf.at[b, pl.ds(r * seq, seq)],
                    out_hbm.at[base + c * BATCHES_PER_CHUNK + r],
                    wsem.at[b])

        def wwait(b):
            for _ in range(BATCHES_PER_CHUNK):
                pltpu.make_async_copy(
                    buf.at[b, pl.ds(0, seq)], out_hbm.at[0],
                    wsem.at[b]).wait()

        for b in range(NBUF):
            gstart(b, b)

        @pl.loop(0, n_groups)
        def _group(g):
            c0 = g * NBUF
            for b in range(NBUF):
                gwait(b)
                wstart(b, c0 + b)

            @pl.when(g < n_groups - 1)
            def _next():
                for b in range(NBUF):
                    wwait(b)
                    gstart(b, c0 + NBUF + b)

        for b in range(NBUF):
            wwait(b)

    return gather_kernel


def kernel(indices, weight):
    n_batch, seq = indices.shape
    dim = weight.shape[-1]
    rows_per_chunk = BATCHES_PER_CHUNK * seq
    idx_grouped = indices.reshape(
        NUM_WORKERS, indices.size // (NUM_WORKERS * rows_per_chunk),
        rows_per_chunk)
    return _make_gather(n_batch, seq, dim)(idx_grouped, weight)


# speedup vs baseline: 2.1289x; 1.0033x over previous
import functools

import jax
import jax.numpy as jnp
from jax import lax
from jax.experimental import pallas as pl
from jax.experimental.pallas import tpu as pltpu
from jax.experimental.pallas import tpu_sc as plsc

NUM_CORES = 2
NUM_SUBCORES = 16
NUM_WORKERS = NUM_CORES * NUM_SUBCORES
BATCHES_PER_CHUNK = 2
NBUF = 4


@functools.lru_cache(maxsize=None)
def _make_gather(n_batch: int, seq: int, dim: int):
    rows_per_chunk = BATCHES_PER_CHUNK * seq
    assert n_batch % (NUM_WORKERS * BATCHES_PER_CHUNK) == 0
    batches_per_w = n_batch // NUM_WORKERS
    n_chunks = batches_per_w // BATCHES_PER_CHUNK
    assert n_chunks % NBUF == 0
    n_groups = n_chunks // NBUF

    mesh = plsc.VectorSubcoreMesh(
        core_axis_name="c", subcore_axis_name="s",
        num_cores=NUM_CORES, num_subcores=NUM_SUBCORES)

    @functools.partial(
        pl.kernel,
        out_type=jax.ShapeDtypeStruct((n_batch, seq, dim), jnp.float32),
        mesh=mesh,
        compiler_params=pltpu.CompilerParams(use_tc_tiling_on_sc=True),
        scratch_types=[
            pltpu.VMEM((n_chunks, rows_per_chunk), jnp.int32),
            pltpu.VMEM((NBUF, rows_per_chunk, dim), jnp.float32),
            pltpu.SemaphoreType.DMA,
            pltpu.SemaphoreType.DMA((NBUF,)),
            pltpu.SemaphoreType.DMA((NBUF,)),
        ],
    )
    def gather_kernel(idx_hbm, table_hbm, out_hbm, idx_v, buf, isem,
                      gsem, wsem):
        wid = lax.axis_index("s") * NUM_CORES + lax.axis_index("c")
        base = wid * batches_per_w

        pltpu.async_copy(idx_hbm.at[wid], idx_v, isem).wait()

        def gstart(b, c):
            pltpu.async_copy(table_hbm.at[idx_v.at[c]], buf.at[b],
                             gsem.at[b])

        def gwait(b):
            pltpu.make_async_copy(
                table_hbm.at[idx_v.at[0]], buf.at[b], gsem.at[b]).wait()

        def wstart(b, c):
            for r in range(BATCHES_PER_CHUNK):
                pltpu.async_copy(
                    buf.at[b, pl.ds(r * seq, seq)],
                    out_hbm.at[base + c * BATCHES_PER_CHUNK + r],
                    wsem.at[b])

        def wwait(b):
            for _ in range(BATCHES_PER_CHUNK):
                pltpu.make_async_copy(
                    buf.at[b, pl.ds(0, seq)], out_hbm.at[0],
                    wsem.at[b]).wait()

        for b in range(NBUF):
            gstart(b, b)

        @pl.loop(0, n_groups)
        def _group(g):
            c0 = g * NBUF
            for b in range(NBUF):
                gwait(b)
                wstart(b, c0 + b)

            @pl.when(g < n_groups - 1)
            def _next():
                for b in range(NBUF):
                    wwait(b)
                    gstart(b, c0 + NBUF + b)

        for b in range(NBUF):
            wwait(b)

    return gather_kernel


def kernel(indices, weight):
    n_batch, seq = indices.shape
    dim = weight.shape[-1]
    rows_per_chunk = BATCHES_PER_CHUNK * seq
    idx_grouped = indices.reshape(
        NUM_WORKERS, indices.size // (NUM_WORKERS * rows_per_chunk),
        rows_per_chunk)
    return _make_gather(n_batch, seq, dim)(idx_grouped, weight)
